# Initial kernel scaffold; baseline (speedup 1.0000x reference)
import functools

import jax
import jax.numpy as jnp
from jax import lax
from jax.experimental import pallas as pl
from jax.experimental.pallas import tpu as pltpu
from jax.experimental.pallas import tpu_sc as plsc

N = 10000
D = 256
NCLS = 40
E = 160000

NP = 10240
NC = 2
NS = 16
K = 128
CH = 80
EP = NS * CH * K
STRIPE = NP // NS
BN = 512
NB = NP // BN

f32 = jnp.float32


_MESH = plsc.VectorSubcoreMesh(core_axis_name="c", subcore_axis_name="s")


@functools.partial(
    pl.kernel,
    out_type=jax.ShapeDtypeStruct((NC, NP, 16), f32),
    mesh=_MESH,
    scratch_types=[
        pltpu.VMEM((CH, K), jnp.int32),
        pltpu.VMEM((K, 16), f32),
        pltpu.VMEM_SHARED((NP, 16), f32),
    ],
)
def _deg_kernel(dst_hbm, ones_hbm, z_hbm, out_hbm, dstv, ones_v, acc):
    c = lax.axis_index("c")
    s = lax.axis_index("s")
    pltpu.sync_copy(dst_hbm.at[s], dstv)
    pltpu.sync_copy(ones_hbm, ones_v)
    pltpu.sync_copy(z_hbm.at[pl.ds(s * STRIPE, STRIPE)],
                    acc.at[pl.ds(s * STRIPE, STRIPE)])
    plsc.subcore_barrier()

    half = CH // NC

    def body(j, carry):
        pltpu.sync_copy(ones_v, acc.at[dstv.at[c * half + j]], add=True)
        return carry

    lax.fori_loop(0, half, body, 0)
    plsc.subcore_barrier()
    pltpu.sync_copy(acc.at[pl.ds(s * STRIPE, STRIPE)],
                    out_hbm.at[c, pl.ds(s * STRIPE, STRIPE)])


@functools.partial(
    pl.kernel,
    out_type=jax.ShapeDtypeStruct((NC, NP, 128), f32),
    mesh=_MESH,
    scratch_types=[
        pltpu.VMEM((CH, K), jnp.int32),
        pltpu.VMEM((CH, K), jnp.int32),
        pltpu.VMEM((K, 128), f32),
        pltpu.VMEM_SHARED((NP, 128), f32),
    ],
)
def _edge_kernel(y_hbm, src_hbm, dst_hbm, z_hbm, out_hbm, srcv, dstv, buf, acc):
    c = lax.axis_index("c")
    s = lax.axis_index("s")
    pltpu.sync_copy(src_hbm.at[c, s], srcv)
    pltpu.sync_copy(dst_hbm.at[s], dstv)
    pltpu.sync_copy(z_hbm.at[pl.ds(s * STRIPE, STRIPE)],
                    acc.at[pl.ds(s * STRIPE, STRIPE)])
    plsc.subcore_barrier()

    def body(j, carry):
        pltpu.sync_copy(y_hbm.at[srcv.at[j]], buf)
        pltpu.sync_copy(buf, acc.at[dstv.at[j]], add=True)
        return carry

    lax.fori_loop(0, CH, body, 0)
    plsc.subcore_barrier()
    pltpu.sync_copy(acc.at[pl.ds(s * STRIPE, STRIPE)],
                    out_hbm.at[c, pl.ds(s * STRIPE, STRIPE)])


def _dinv_from(deg_ref):
    d = deg_ref[0, :, 0] + deg_ref[1, :, 0] + 1.0
    return lax.rsqrt(d)[:, None]


def _mm_in_body(x_ref, w_ref, deg_ref, o_ref):
    dinv = _dinv_from(deg_ref)
    y = jnp.dot(x_ref[...], w_ref[...], preferred_element_type=f32)
    o_ref[...] = (dinv * y)[None]


def _mid_body(s_ref, y_ref, deg_ref, b_ref, w_ref, o_ref):
    dinv = _dinv_from(deg_ref)
    sf = jnp.concatenate([s_ref[0], s_ref[1]], axis=1)
    yf = jnp.concatenate([y_ref[0], y_ref[1]], axis=1)
    h = jnp.maximum(dinv * (sf + yf) + b_ref[...], 0.0)
    y2 = dinv * jnp.dot(h, w_ref[...], preferred_element_type=f32)
    o_ref[0] = y2[:, :128]
    o_ref[1] = y2[:, 128:]


def _head_body(s_ref, y_ref, deg_ref, b2_ref, w3_ref, b3_ref, w4_ref,
               b4_ref, o_ref):
    dinv = _dinv_from(deg_ref)
    sf = jnp.concatenate([s_ref[0], s_ref[1]], axis=1)
    yf = jnp.concatenate([y_ref[0], y_ref[1]], axis=1)
    h2 = jnp.maximum(dinv * (sf + yf) + b2_ref[...], 0.0)
    h3 = jnp.maximum(
        jnp.dot(h2, w3_ref[...], preferred_element_type=f32) + b3_ref[...], 0.0)
    o_ref[...] = jnp.dot(h3, w4_ref[...], preferred_element_type=f32) + b4_ref[...]


def _mm_in(x, w, degraw):
    return pl.pallas_call(
        _mm_in_body,
        grid=(NB, 2),
        in_specs=[
            pl.BlockSpec((BN, D), lambda i, h: (i, 0)),
            pl.BlockSpec((D, 128), lambda i, h: (0, h)),
            pl.BlockSpec((2, BN, 16), lambda i, h: (0, i, 0)),
        ],
        out_specs=pl.BlockSpec((1, BN, 128), lambda i, h: (h, i, 0)),
        out_shape=jax.ShapeDtypeStruct((2, NP, 128), f32),
    )(x, w, degraw)


def _mid(sacc, y, degraw, b, w):
    return pl.pallas_call(
        _mid_body,
        grid=(NB,),
        in_specs=[
            pl.BlockSpec((2, BN, 128), lambda i: (0, i, 0)),
            pl.BlockSpec((2, BN, 128), lambda i: (0, i, 0)),
            pl.BlockSpec((2, BN, 16), lambda i: (0, i, 0)),
            pl.BlockSpec((1, D), lambda i: (0, 0)),
            pl.BlockSpec((D, D), lambda i: (0, 0)),
        ],
        out_specs=pl.BlockSpec((2, BN, 128), lambda i: (0, i, 0)),
        out_shape=jax.ShapeDtypeStruct((2, NP, 128), f32),
    )(sacc, y, degraw, b, w)


def _head(sacc, y, degraw, b2, w3, b3, w4p, b4p):
    return pl.pallas_call(
        _head_body,
        grid=(NB,),
        in_specs=[
            pl.BlockSpec((2, BN, 128), lambda i: (0, i, 0)),
            pl.BlockSpec((2, BN, 128), lambda i: (0, i, 0)),
            pl.BlockSpec((2, BN, 16), lambda i: (0, i, 0)),
            pl.BlockSpec((1, D), lambda i: (0, 0)),
            pl.BlockSpec((D, D), lambda i: (0, 0)),
            pl.BlockSpec((1, D), lambda i: (0, 0)),
            pl.BlockSpec((D, 128), lambda i: (0, 0)),
            pl.BlockSpec((1, 128), lambda i: (0, 0)),
        ],
        out_specs=pl.BlockSpec((BN, 128), lambda i: (i, 0)),
        out_shape=jax.ShapeDtypeStruct((NP, 128), f32),
    )(sacc, y, degraw, b2, w3, b3, w4p, b4p)


def kernel(x, edge_index, W1, b1, W2, b2, W3, b3, W4, b4):
    x = x.astype(f32)
    src = edge_index[0].astype(jnp.int32)
    dst = edge_index[1].astype(jnp.int32)

    pad = EP - E
    padv = jnp.full((pad,), NP - 1, jnp.int32)
    src_p = jnp.concatenate([src, padv])
    dst_p = jnp.concatenate([dst, padv])
    src_both = jnp.stack([src_p, src_p + NP]).reshape(NC, NS, CH, K)
    dst_t = dst_p.reshape(NS, CH, K)

    ones16 = jnp.ones((K, 16), f32)
    z16 = jnp.zeros((NP, 16), f32)
    z128 = jnp.zeros((NP, 128), f32)
    xp = jnp.pad(x, ((0, NP - N), (0, 0)))
    b1r = b1.reshape(1, D)
    b2r = b2.reshape(1, D)
    b3r = b3.reshape(1, D)
    w4p = jnp.pad(W4, ((0, 0), (0, 128 - NCLS)))
    b4p = jnp.pad(b4, (0, 128 - NCLS)).reshape(1, 128)

    degraw = _deg_kernel(dst_t, ones16, z16)

    y1 = _mm_in(xp, W1, degraw)
    s1 = _edge_kernel(y1.reshape(NC * NP, 128), src_both, dst_t, z128)
    y2 = _mid(s1, y1, degraw, b1r, W2)
    s2 = _edge_kernel(y2.reshape(NC * NP, 128), src_both, dst_t, z128)
    out = _head(s2, y2, degraw, b2r, W3, b3r, w4p, b4p)
    return out[:N, :NCLS]


# speedup vs baseline: 6.2814x; 6.2814x over previous
import functools

import jax
import jax.numpy as jnp
from jax import lax
from jax.experimental import pallas as pl
from jax.experimental.pallas import tpu as pltpu
from jax.experimental.pallas import tpu_sc as plsc

N = 10000
D = 256
NCLS = 40
E = 160000

NP = 10240
NC = 2
NS = 16
K = 128
CH = 80
EP = NS * CH * K
STRIPE = NP // NS
BN = 512
NB = NP // BN

f32 = jnp.float32


@functools.cache
def _build_deg_kernel():
    mesh = plsc.VectorSubcoreMesh(core_axis_name="c", subcore_axis_name="s")

    @functools.partial(
        pl.kernel,
        out_type=jax.ShapeDtypeStruct((NC, NP, 128), f32),
        mesh=mesh,
        scratch_types=[
            pltpu.VMEM((CH, K), jnp.int32),
            pltpu.VMEM((K, 128), f32),
            pltpu.VMEM_SHARED((NP, 128), f32),
        ],
    )
    def deg_kernel(dst_hbm, ones_hbm, z_hbm, out_hbm, dstv, ones_v, acc):
        c = lax.axis_index("c")
        s = lax.axis_index("s")
        pltpu.sync_copy(dst_hbm.at[s], dstv)
        pltpu.sync_copy(ones_hbm, ones_v)
        pltpu.sync_copy(z_hbm.at[pl.ds(s * STRIPE, STRIPE)],
                        acc.at[pl.ds(s * STRIPE, STRIPE)])
        plsc.subcore_barrier()

        half = CH // NC

        def body(j, carry):
            pltpu.sync_copy(ones_v, acc.at[dstv.at[c * half + j]], add=True)
            return carry

        lax.fori_loop(0, half, body, 0)
        plsc.subcore_barrier()
        pltpu.sync_copy(acc.at[pl.ds(s * STRIPE, STRIPE)],
                        out_hbm.at[c, pl.ds(s * STRIPE, STRIPE)])

    return deg_kernel


@functools.cache
def _build_edge_kernel():
    mesh = plsc.VectorSubcoreMesh(core_axis_name="c", subcore_axis_name="s")

    @functools.partial(
        pl.kernel,
        out_type=jax.ShapeDtypeStruct((NC, NP, 128), f32),
        mesh=mesh,
        scratch_types=[
            pltpu.VMEM((CH, K), jnp.int32),
            pltpu.VMEM((CH, K), jnp.int32),
            pltpu.VMEM((K, 128), f32),
            pltpu.VMEM_SHARED((NP, 128), f32),
        ],
    )
    def edge_kernel(y_hbm, src_hbm, dst_hbm, z_hbm, out_hbm,
                    srcv, dstv, buf, acc):
        c = lax.axis_index("c")
        s = lax.axis_index("s")
        pltpu.sync_copy(src_hbm.at[c, s], srcv)
        pltpu.sync_copy(dst_hbm.at[s], dstv)
        pltpu.sync_copy(z_hbm.at[pl.ds(s * STRIPE, STRIPE)],
                        acc.at[pl.ds(s * STRIPE, STRIPE)])
        plsc.subcore_barrier()

        def body(j, carry):
            pltpu.sync_copy(y_hbm.at[srcv.at[j]], buf)
            pltpu.sync_copy(buf, acc.at[dstv.at[j]], add=True)
            return carry

        lax.fori_loop(0, CH, body, 0)
        plsc.subcore_barrier()
        pltpu.sync_copy(acc.at[pl.ds(s * STRIPE, STRIPE)],
                        out_hbm.at[c, pl.ds(s * STRIPE, STRIPE)])

    return edge_kernel


def _deg_kernel(dst_t, ones16, z16):
    return _build_deg_kernel()(dst_t, ones16, z16)


def _edge_kernel(y2d, src_both, dst_t, z128):
    return _build_edge_kernel()(y2d, src_both, dst_t, z128)


def _dinv_from(deg_ref):
    d = deg_ref[0, :, 0] + deg_ref[1, :, 0] + 1.0
    return lax.rsqrt(d)[:, None]


def _mm_in_body(x_ref, w_ref, deg_ref, o_ref):
    dinv = _dinv_from(deg_ref)
    y = jnp.dot(x_ref[...], w_ref[...], preferred_element_type=f32)
    o_ref[...] = (dinv * y)[None]


def _mid_body(s_ref, y_ref, deg_ref, b_ref, w_ref, o_ref):
    dinv = _dinv_from(deg_ref)
    sf = jnp.concatenate([s_ref[0], s_ref[1]], axis=1)
    yf = jnp.concatenate([y_ref[0], y_ref[1]], axis=1)
    h = jnp.maximum(dinv * (sf + yf) + b_ref[...], 0.0)
    y2 = dinv * jnp.dot(h, w_ref[...], preferred_element_type=f32)
    o_ref[0] = y2[:, :128]
    o_ref[1] = y2[:, 128:]


def _head_body(s_ref, y_ref, deg_ref, b2_ref, w3_ref, b3_ref, w4_ref,
               b4_ref, o_ref):
    dinv = _dinv_from(deg_ref)
    sf = jnp.concatenate([s_ref[0], s_ref[1]], axis=1)
    yf = jnp.concatenate([y_ref[0], y_ref[1]], axis=1)
    h2 = jnp.maximum(dinv * (sf + yf) + b2_ref[...], 0.0)
    h3 = jnp.maximum(
        jnp.dot(h2, w3_ref[...], preferred_element_type=f32) + b3_ref[...], 0.0)
    o_ref[...] = jnp.dot(h3, w4_ref[...], preferred_element_type=f32) + b4_ref[...]


def _mm_in(x, w, degraw):
    return pl.pallas_call(
        _mm_in_body,
        grid=(NB, 2),
        in_specs=[
            pl.BlockSpec((BN, D), lambda i, h: (i, 0)),
            pl.BlockSpec((D, 128), lambda i, h: (0, h)),
            pl.BlockSpec((2, BN, 128), lambda i, h: (0, i, 0)),
        ],
        out_specs=pl.BlockSpec((1, BN, 128), lambda i, h: (h, i, 0)),
        out_shape=jax.ShapeDtypeStruct((2, NP, 128), f32),
    )(x, w, degraw)


def _mid(sacc, y, degraw, b, w):
    return pl.pallas_call(
        _mid_body,
        grid=(NB,),
        in_specs=[
            pl.BlockSpec((2, BN, 128), lambda i: (0, i, 0)),
            pl.BlockSpec((2, BN, 128), lambda i: (0, i, 0)),
            pl.BlockSpec((2, BN, 128), lambda i: (0, i, 0)),
            pl.BlockSpec((1, D), lambda i: (0, 0)),
            pl.BlockSpec((D, D), lambda i: (0, 0)),
        ],
        out_specs=pl.BlockSpec((2, BN, 128), lambda i: (0, i, 0)),
        out_shape=jax.ShapeDtypeStruct((2, NP, 128), f32),
    )(sacc, y, degraw, b, w)


def _head(sacc, y, degraw, b2, w3, b3, w4p, b4p):
    return pl.pallas_call(
        _head_body,
        grid=(NB,),
        in_specs=[
            pl.BlockSpec((2, BN, 128), lambda i: (0, i, 0)),
            pl.BlockSpec((2, BN, 128), lambda i: (0, i, 0)),
            pl.BlockSpec((2, BN, 128), lambda i: (0, i, 0)),
            pl.BlockSpec((1, D), lambda i: (0, 0)),
            pl.BlockSpec((D, D), lambda i: (0, 0)),
            pl.BlockSpec((1, D), lambda i: (0, 0)),
            pl.BlockSpec((D, 128), lambda i: (0, 0)),
            pl.BlockSpec((1, 128), lambda i: (0, 0)),
        ],
        out_specs=pl.BlockSpec((BN, 128), lambda i: (i, 0)),
        out_shape=jax.ShapeDtypeStruct((NP, 128), f32),
    )(sacc, y, degraw, b2, w3, b3, w4p, b4p)


def kernel(x, edge_index, W1, b1, W2, b2, W3, b3, W4, b4):
    x = x.astype(f32)
    src = edge_index[0].astype(jnp.int32)
    dst = edge_index[1].astype(jnp.int32)

    pad = EP - E
    padv = jnp.full((pad,), NP - 1, jnp.int32)
    src_p = jnp.concatenate([src, padv])
    dst_p = jnp.concatenate([dst, padv])
    src_both = jnp.stack([src_p, src_p + NP]).reshape(NC, NS, CH, K)
    dst_t = dst_p.reshape(NS, CH, K)

    ones128 = jnp.ones((K, 128), f32)
    z128 = jnp.zeros((NP, 128), f32)
    xp = jnp.pad(x, ((0, NP - N), (0, 0)))
    b1r = b1.reshape(1, D)
    b2r = b2.reshape(1, D)
    b3r = b3.reshape(1, D)
    w4p = jnp.pad(W4, ((0, 0), (0, 128 - NCLS)))
    b4p = jnp.pad(b4, (0, 128 - NCLS)).reshape(1, 128)

    degraw = _deg_kernel(dst_t, ones128, z128)

    y1 = _mm_in(xp, W1, degraw)
    s1 = _edge_kernel(y1.reshape(NC * NP, 128), src_both, dst_t, z128)
    y2 = _mid(s1, y1, degraw, b1r, W2)
    s2 = _edge_kernel(y2.reshape(NC * NP, 128), src_both, dst_t, z128)
    out = _head(s2, y2, degraw, b2r, W3, b3r, w4p, b4p)
    return out[:N, :NCLS]
